# baseline (speedup 1.0000x reference)
import functools

import jax
import jax.numpy as jnp
from jax import lax
from jax.experimental import pallas as pl
from jax.experimental.pallas import tpu as pltpu
from jax.experimental.pallas import tpu_sc as plsc

N = 10000
C = 128
O = 128
K = 32

NC = 2
NS = 16
NW = NC * NS
NPW = 320
NPAD = NW * NPW
G = 2
R = 2 * G * K
NG = NPW // G
D = 4
LANES = 16
NCH = O // LANES


def _mm_body(x_ref, wa_ref, wb_ref, a_ref, b_ref):
    xb = x_ref[...]
    a_ref[...] = jnp.dot(xb, wa_ref[...], preferred_element_type=jnp.float32)
    b_ref[...] = jnp.dot(xb, wb_ref[...], preferred_element_type=jnp.float32)


def _node_tables(xt, wa, wb):
    blk = 2000
    grid = (N // blk,)
    return pl.pallas_call(
        _mm_body,
        grid=grid,
        in_specs=[
            pl.BlockSpec((blk, C), lambda i: (i, 0)),
            pl.BlockSpec((C, O), lambda i: (0, 0)),
            pl.BlockSpec((C, O), lambda i: (0, 0)),
        ],
        out_specs=[
            pl.BlockSpec((blk, O), lambda i: (i, 0)),
            pl.BlockSpec((blk, O), lambda i: (i, 0)),
        ],
        out_shape=[
            jax.ShapeDtypeStruct((N, O), jnp.float32),
            jax.ShapeDtypeStruct((N, O), jnp.float32),
        ],
    )(xt, wa, wb)


def _tree_max(vs):
    while len(vs) > 1:
        nxt = [jnp.maximum(vs[i], vs[i + 1]) for i in range(0, len(vs) - 1, 2)]
        if len(vs) % 2:
            nxt.append(vs[-1])
        vs = nxt
    return vs[0]


def _sc_body(t_hbm, idx_hbm, bias_hbm, out_hbm,
             idx_v, bias_v, out_v, bufs, sems):
    wid = lax.axis_index("s") * NC + lax.axis_index("c")

    pltpu.sync_copy(idx_hbm.at[wid], idx_v)
    pltpu.sync_copy(bias_hbm, bias_v)

    def start(g, buf, sem):
        pltpu.async_copy(t_hbm.at[idx_v.at[g]], buf, sem)

    def wait(g, buf, sem):
        pltpu.make_async_copy(t_hbm.at[idx_v.at[g]], buf, sem).wait()

    def compute(g, buf):
        for j in range(G):
            base = 2 * j * K
            for c in range(NCH):
                sl = pl.ds(c * LANES, LANES)
                vs = [buf[base + 2 * r, sl] + buf[base + 2 * r + 1, sl]
                      for r in range(K)]
                m = _tree_max(vs)
                m = jnp.maximum(m + bias_v[sl], 0.0)
                out_v[pl.ds((g * G + j) * O + c * LANES, LANES)] = m

    for d in range(D):
        start(d, bufs[d], sems[d])

    def body(gd, carry):
        for d in range(D):
            g = gd * D + d
            wait(g, bufs[d], sems[d])
            compute(g, bufs[d])
            start(g + D, bufs[d], sems[d])
        return carry

    lax.fori_loop(0, NG // D, body, 0)

    for d in range(D):
        wait(NG + d, bufs[d], sems[d])

    pltpu.sync_copy(out_v, out_hbm.at[wid])


@functools.partial(
    pl.kernel,
    out_type=jax.ShapeDtypeStruct((NW, NPW * O), jnp.float32),
    mesh=plsc.VectorSubcoreMesh(core_axis_name="c", subcore_axis_name="s"),
    scratch_types=(
        [
            pltpu.VMEM((NG + D, R), jnp.int32),
            pltpu.VMEM((O,), jnp.float32),
            pltpu.VMEM((NPW * O,), jnp.float32),
        ]
        + [pltpu.VMEM((R, O), jnp.float32) for _ in range(D)]
        + [pltpu.SemaphoreType.DMA for _ in range(D)]
    ),
)
def _sc_gather_max(t_hbm, idx_hbm, bias_hbm, out_hbm, idx_v, bias_v, out_v,
                   *rest):
    _sc_body(t_hbm, idx_hbm, bias_hbm, out_hbm,
             idx_v, bias_v, out_v, rest[:D], rest[D:])


def kernel(x, edge_index, W, b):
    xt = x[0, :, :, 0].T
    w1 = W[:, :C]
    w2 = W[:, C:]
    wa = (w1 - w2).T
    wb = w2.T

    a_tab, b_tab = _node_tables(xt, wa, wb)
    t_tab = jnp.stack([a_tab, b_tab], axis=1).reshape(2 * N, O)

    ei = edge_index.astype(jnp.int32).reshape(2, N * K)
    pad = NPAD * K - N * K
    idx1 = jnp.pad(2 * ei[1], (0, pad))
    idx0 = jnp.pad(2 * ei[0] + 1, (0, pad))
    idx = jnp.stack([idx1, idx0], axis=1).reshape(NW, NG, R)
    zrow = jnp.zeros((NW, D, R), jnp.int32)
    idx = jnp.concatenate([idx, zrow], axis=1)

    out = _sc_gather_max(t_tab, idx, b)
    out = out.reshape(NPAD, O)[:N].T
    return out[None]


# speedup vs baseline: 1.7425x; 1.7425x over previous
import functools

import jax
import jax.numpy as jnp
from jax import lax
from jax.experimental import pallas as pl
from jax.experimental.pallas import tpu as pltpu
from jax.experimental.pallas import tpu_sc as plsc

N = 10000
C = 128
O = 128
K = 32

NC = 2
NS = 16
NW = NC * NS
NPW = 318
NPAD = NW * NPW
G = 2
R = G * K
NG = NPW // G
D = 3
LANES = 16
NCH = O // LANES


def _mm_body(x_ref, wa_ref, wb_ref, a_ref, b_ref):
    xb = x_ref[...]
    a_ref[...] = jnp.dot(xb, wa_ref[...], preferred_element_type=jnp.float32)
    b_ref[...] = jnp.dot(xb, wb_ref[...], preferred_element_type=jnp.float32)


def _node_tables(xt, wa, wb):
    blk = 2000
    grid = (N // blk,)
    return pl.pallas_call(
        _mm_body,
        grid=grid,
        in_specs=[
            pl.BlockSpec((blk, C), lambda i: (i, 0)),
            pl.BlockSpec((C, O), lambda i: (0, 0)),
            pl.BlockSpec((C, O), lambda i: (0, 0)),
        ],
        out_specs=[
            pl.BlockSpec((blk, O), lambda i: (i, 0)),
            pl.BlockSpec((blk, O), lambda i: (i, 0)),
        ],
        out_shape=[
            jax.ShapeDtypeStruct((N, O), jnp.float32),
            jax.ShapeDtypeStruct((N, O), jnp.float32),
        ],
    )(xt, wa, wb)


def _tree_max(vs):
    while len(vs) > 1:
        nxt = [jnp.maximum(vs[i], vs[i + 1]) for i in range(0, len(vs) - 1, 2)]
        if len(vs) % 2:
            nxt.append(vs[-1])
        vs = nxt
    return vs[0]


def _sc_body(a_hbm, b_hbm, idx1_hbm, idx0_hbm, bias_hbm, out_hbm,
             idx1_v, idx0_v, bias_v, obufs, bufs_a, bufs_b,
             sems_a, sems_b, osems):
    wid = lax.axis_index("s") * NC + lax.axis_index("c")

    pltpu.sync_copy(idx1_hbm.at[wid], idx1_v)
    pltpu.sync_copy(idx0_hbm.at[wid], idx0_v)
    pltpu.sync_copy(bias_hbm, bias_v)

    def start(g, idx_v, table, buf, sem):
        return pltpu.async_copy(table.at[idx_v.at[g]], buf, sem)

    def wait(g, idx_v, table, buf, sem):
        pltpu.make_async_copy(table.at[idx_v.at[g]], buf, sem).wait()

    def compute(g, buf_a, buf_b, obuf):
        for j in range(G):
            base = j * K
            for c in range(NCH):
                sl = pl.ds(c * LANES, LANES)
                acc = [buf_a[base + p, sl] + buf_b[base + p, sl]
                       for p in range(4)]
                for r in range(4, K):
                    acc[r % 4] = jnp.maximum(
                        acc[r % 4], buf_a[base + r, sl] + buf_b[base + r, sl])
                m = jnp.maximum(jnp.maximum(acc[0], acc[1]),
                                jnp.maximum(acc[2], acc[3]))
                m = jnp.maximum(m + bias_v[sl], 0.0)
                obuf[pl.ds(j * O + c * LANES, LANES)] = m

    def out_slice(g):
        return out_hbm.at[wid, pl.ds(g * (G * O), G * O)]

    for d in range(D):
        start(d, idx1_v, a_hbm, bufs_a[d], sems_a[d])
        start(d, idx0_v, b_hbm, bufs_b[d], sems_b[d])

    def body(gd, carry):
        for d in range(D):
            g = gd * D + d
            wait(g, idx1_v, a_hbm, bufs_a[d], sems_a[d])
            wait(g, idx0_v, b_hbm, bufs_b[d], sems_b[d])

            @pl.when(g >= D)
            def _():
                pltpu.make_async_copy(obufs[d], out_slice(g - D),
                                      osems[d]).wait()

            compute(g, bufs_a[d], bufs_b[d], obufs[d])
            pltpu.async_copy(obufs[d], out_slice(g), osems[d])
            start(g + D, idx1_v, a_hbm, bufs_a[d], sems_a[d])
            start(g + D, idx0_v, b_hbm, bufs_b[d], sems_b[d])
        return carry

    lax.fori_loop(0, NG // D, body, 0)

    for d in range(D):
        wait(NG + d, idx1_v, a_hbm, bufs_a[d], sems_a[d])
        wait(NG + d, idx0_v, b_hbm, bufs_b[d], sems_b[d])
        pltpu.make_async_copy(obufs[d], out_slice(NG - D + d),
                              osems[d]).wait()


@functools.partial(
    pl.kernel,
    out_type=jax.ShapeDtypeStruct((NW, NPW * O), jnp.float32),
    mesh=plsc.VectorSubcoreMesh(core_axis_name="c", subcore_axis_name="s"),
    scratch_types=(
        [
            pltpu.VMEM((NG + D, R), jnp.int32),
            pltpu.VMEM((NG + D, R), jnp.int32),
            pltpu.VMEM((O,), jnp.float32),
        ]
        + [pltpu.VMEM((G * O,), jnp.float32) for _ in range(D)]
        + [pltpu.VMEM((R, O), jnp.float32) for _ in range(2 * D)]
        + [pltpu.SemaphoreType.DMA for _ in range(3 * D)]
    ),
)
def _sc_gather_max(a_hbm, b_hbm, idx1_hbm, idx0_hbm, bias_hbm, out_hbm,
                   idx1_v, idx0_v, bias_v, *rest):
    _sc_body(a_hbm, b_hbm, idx1_hbm, idx0_hbm, bias_hbm, out_hbm,
             idx1_v, idx0_v, bias_v,
             rest[:D], rest[D:2 * D], rest[2 * D:3 * D],
             rest[3 * D:4 * D], rest[4 * D:5 * D], rest[5 * D:])


def kernel(x, edge_index, W, b):
    xt = x[0, :, :, 0].T
    w1 = W[:, :C]
    w2 = W[:, C:]
    wa = (w1 - w2).T
    wb = w2.T

    a_tab, b_tab = _node_tables(xt, wa, wb)

    ei = edge_index.astype(jnp.int32).reshape(2, N * K)
    pad = NPAD * K - N * K
    idx1 = jnp.pad(ei[1], (0, pad)).reshape(NW, NG, R)
    idx0 = jnp.pad(ei[0], (0, pad)).reshape(NW, NG, R)
    zrow = jnp.zeros((NW, D, R), jnp.int32)
    idx1 = jnp.concatenate([idx1, zrow], axis=1)
    idx0 = jnp.concatenate([idx0, zrow], axis=1)

    out = _sc_gather_max(a_tab, b_tab, idx1, idx0, b)
    out = out.reshape(NPAD, O)[:N].T
    return out[None]


# speedup vs baseline: 1.9992x; 1.1473x over previous
import functools

import jax
import jax.numpy as jnp
from jax import lax
from jax.experimental import pallas as pl
from jax.experimental.pallas import tpu as pltpu
from jax.experimental.pallas import tpu_sc as plsc

N = 10000
C = 128
O = 128
K = 32

NC = 2
NS = 16
NW = NC * NS
NPW = 318
NPAD = NW * NPW
G = 2
R = G * K
NG = NPW // G
D = 3
LANES = 16
NCH = O // LANES


def _mm_body(x_ref, wa_ref, wb_ref, a_ref, b_ref):
    xb = x_ref[...]
    a_ref[...] = jnp.dot(xb, wa_ref[...], preferred_element_type=jnp.float32)
    b_ref[...] = jnp.dot(xb, wb_ref[...], preferred_element_type=jnp.float32)


def _node_tables(xt, wa, wb):
    blk = 2000
    grid = (N // blk,)
    return pl.pallas_call(
        _mm_body,
        grid=grid,
        in_specs=[
            pl.BlockSpec((blk, C), lambda i: (i, 0)),
            pl.BlockSpec((C, O), lambda i: (0, 0)),
            pl.BlockSpec((C, O), lambda i: (0, 0)),
        ],
        out_specs=[
            pl.BlockSpec((blk, O), lambda i: (i, 0)),
            pl.BlockSpec((blk, O), lambda i: (i, 0)),
        ],
        out_shape=[
            jax.ShapeDtypeStruct((N, O), jnp.float32),
            jax.ShapeDtypeStruct((N, O), jnp.float32),
        ],
    )(xt, wa, wb)


def _tree_max(vs):
    while len(vs) > 1:
        nxt = [jnp.maximum(vs[i], vs[i + 1]) for i in range(0, len(vs) - 1, 2)]
        if len(vs) % 2:
            nxt.append(vs[-1])
        vs = nxt
    return vs[0]


def _sc_body(a_hbm, b_hbm, idx1_hbm, idx0_hbm, bias_hbm, out_hbm,
             idx1_v, idx0_v, bias_v, obufs, bufs_a, bufs_b,
             sems_a, sems_b, osems):
    wid = lax.axis_index("s") * NC + lax.axis_index("c")

    pltpu.sync_copy(idx1_hbm.at[wid], idx1_v)
    pltpu.sync_copy(idx0_hbm.at[wid], idx0_v)
    pltpu.sync_copy(bias_hbm, bias_v)

    def start(g, idx_v, table, buf, sem):
        return pltpu.async_copy(table.at[idx_v.at[g]], buf, sem)

    def wait(g, idx_v, table, buf, sem):
        pltpu.make_async_copy(table.at[idx_v.at[g]], buf, sem).wait()

    def compute(g, buf_a, buf_b, obuf):
        for j in range(G):
            base = j * K

            def row_body(r, accs):
                return tuple(
                    jnp.maximum(accs[c],
                                buf_a[base + r, pl.ds(c * LANES, LANES)]
                                + buf_b[base + r, pl.ds(c * LANES, LANES)])
                    for c in range(NCH))

            init = tuple(buf_a[base, pl.ds(c * LANES, LANES)]
                         + buf_b[base, pl.ds(c * LANES, LANES)]
                         for c in range(NCH))
            accs = lax.fori_loop(1, K, row_body, init)
            for c in range(NCH):
                sl = pl.ds(c * LANES, LANES)
                m = jnp.maximum(accs[c] + bias_v[sl], 0.0)
                obuf[pl.ds(j * O + c * LANES, LANES)] = m

    def out_slice(g):
        return out_hbm.at[wid, pl.ds(g * (G * O), G * O)]

    for d in range(D):
        start(d, idx1_v, a_hbm, bufs_a[d], sems_a[d])
        start(d, idx0_v, b_hbm, bufs_b[d], sems_b[d])

    def body(gd, carry):
        for d in range(D):
            g = gd * D + d
            wait(g, idx1_v, a_hbm, bufs_a[d], sems_a[d])
            wait(g, idx0_v, b_hbm, bufs_b[d], sems_b[d])

            @pl.when(g >= D)
            def _():
                pltpu.make_async_copy(obufs[d], out_slice(g - D),
                                      osems[d]).wait()

            compute(g, bufs_a[d], bufs_b[d], obufs[d])
            pltpu.async_copy(obufs[d], out_slice(g), osems[d])
            start(g + D, idx1_v, a_hbm, bufs_a[d], sems_a[d])
            start(g + D, idx0_v, b_hbm, bufs_b[d], sems_b[d])
        return carry

    lax.fori_loop(0, NG // D, body, 0)

    for d in range(D):
        wait(NG + d, idx1_v, a_hbm, bufs_a[d], sems_a[d])
        wait(NG + d, idx0_v, b_hbm, bufs_b[d], sems_b[d])
        pltpu.make_async_copy(obufs[d], out_slice(NG - D + d),
                              osems[d]).wait()


@functools.partial(
    pl.kernel,
    out_type=jax.ShapeDtypeStruct((NW, NPW * O), jnp.float32),
    mesh=plsc.VectorSubcoreMesh(core_axis_name="c", subcore_axis_name="s"),
    scratch_types=(
        [
            pltpu.VMEM((NG + D, R), jnp.int32),
            pltpu.VMEM((NG + D, R), jnp.int32),
            pltpu.VMEM((O,), jnp.float32),
        ]
        + [pltpu.VMEM((G * O,), jnp.float32) for _ in range(D)]
        + [pltpu.VMEM((R, O), jnp.float32) for _ in range(2 * D)]
        + [pltpu.SemaphoreType.DMA for _ in range(3 * D)]
    ),
)
def _sc_gather_max(a_hbm, b_hbm, idx1_hbm, idx0_hbm, bias_hbm, out_hbm,
                   idx1_v, idx0_v, bias_v, *rest):
    _sc_body(a_hbm, b_hbm, idx1_hbm, idx0_hbm, bias_hbm, out_hbm,
             idx1_v, idx0_v, bias_v,
             rest[:D], rest[D:2 * D], rest[2 * D:3 * D],
             rest[3 * D:4 * D], rest[4 * D:5 * D], rest[5 * D:])


def kernel(x, edge_index, W, b):
    xt = x[0, :, :, 0].T
    w1 = W[:, :C]
    w2 = W[:, C:]
    wa = (w1 - w2).T
    wb = w2.T

    a_tab, b_tab = _node_tables(xt, wa, wb)

    ei = edge_index.astype(jnp.int32).reshape(2, N * K)
    pad = NPAD * K - N * K
    idx1 = jnp.pad(ei[1], (0, pad)).reshape(NW, NG, R)
    idx0 = jnp.pad(ei[0], (0, pad)).reshape(NW, NG, R)
    zrow = jnp.zeros((NW, D, R), jnp.int32)
    idx1 = jnp.concatenate([idx1, zrow], axis=1)
    idx0 = jnp.concatenate([idx0, zrow], axis=1)

    out = _sc_gather_max(a_tab, b_tab, idx1, idx0, b)
    out = out.reshape(NPAD, O)[:N].T
    return out[None]


# speedup vs baseline: 2.0985x; 1.0497x over previous
import functools

import jax
import jax.numpy as jnp
from jax import lax
from jax.experimental import pallas as pl
from jax.experimental.pallas import tpu as pltpu
from jax.experimental.pallas import tpu_sc as plsc

N = 10000
C = 128
O = 128
K = 32

NC = 2
NS = 16
NW = NC * NS
NPW = 320
NPAD = NW * NPW
G = 2
R = G * K
NG = NPW // G
LANES = 16
NCH = O // LANES


def _mm_body(x_ref, wa_ref, wb_ref, a_ref, b_ref):
    xb = x_ref[...]
    a_ref[...] = jnp.dot(xb, wa_ref[...], preferred_element_type=jnp.float32)
    b_ref[...] = jnp.dot(xb, wb_ref[...], preferred_element_type=jnp.float32)


def _node_tables(xt, wa, wb):
    blk = 2000
    grid = (N // blk,)
    return pl.pallas_call(
        _mm_body,
        grid=grid,
        in_specs=[
            pl.BlockSpec((blk, C), lambda i: (i, 0)),
            pl.BlockSpec((C, O), lambda i: (0, 0)),
            pl.BlockSpec((C, O), lambda i: (0, 0)),
        ],
        out_specs=[
            pl.BlockSpec((blk, O), lambda i: (i, 0)),
            pl.BlockSpec((blk, O), lambda i: (i, 0)),
        ],
        out_shape=[
            jax.ShapeDtypeStruct((N, O), jnp.float32),
            jax.ShapeDtypeStruct((N, O), jnp.float32),
        ],
    )(xt, wa, wb)


def _tree_max(vs):
    while len(vs) > 1:
        nxt = [jnp.maximum(vs[i], vs[i + 1]) for i in range(0, len(vs) - 1, 2)]
        if len(vs) % 2:
            nxt.append(vs[-1])
        vs = nxt
    return vs[0]


def _sc_body(a_hbm, b_hbm, idx1_hbm, idx0_hbm, bias_hbm, out_hbm,
             idx1_v, idx0_v, bias_v, out_v,
             buf_a0, buf_b0, buf_a1, buf_b1,
             sem_a0, sem_b0, sem_a1, sem_b1):
    wid = lax.axis_index("s") * NC + lax.axis_index("c")

    pltpu.sync_copy(idx1_hbm.at[wid], idx1_v)
    pltpu.sync_copy(idx0_hbm.at[wid], idx0_v)
    pltpu.sync_copy(bias_hbm, bias_v)

    def start(g, idx_v, table, buf, sem):
        return pltpu.async_copy(table.at[idx_v.at[g]], buf, sem)

    def wait(g, idx_v, table, buf, sem):
        pltpu.make_async_copy(table.at[idx_v.at[g]], buf, sem).wait()

    def compute(g, buf_a, buf_b):
        for j in range(G):
            base = j * K
            for c in range(NCH):
                sl = pl.ds(c * LANES, LANES)
                vs = [buf_a[base + r, sl] + buf_b[base + r, sl]
                      for r in range(K)]
                m = _tree_max(vs)
                m = jnp.maximum(m + bias_v[sl], 0.0)
                out_v[pl.ds((g * G + j) * O + c * LANES, LANES)] = m

    start(0, idx1_v, a_hbm, buf_a0, sem_a0)
    start(0, idx0_v, b_hbm, buf_b0, sem_b0)

    def body(g2, carry):
        g0 = 2 * g2
        g1 = g0 + 1
        start(g1, idx1_v, a_hbm, buf_a1, sem_a1)
        start(g1, idx0_v, b_hbm, buf_b1, sem_b1)
        wait(g0, idx1_v, a_hbm, buf_a0, sem_a0)
        wait(g0, idx0_v, b_hbm, buf_b0, sem_b0)
        compute(g0, buf_a0, buf_b0)
        start(g0 + 2, idx1_v, a_hbm, buf_a0, sem_a0)
        start(g0 + 2, idx0_v, b_hbm, buf_b0, sem_b0)
        wait(g1, idx1_v, a_hbm, buf_a1, sem_a1)
        wait(g1, idx0_v, b_hbm, buf_b1, sem_b1)
        compute(g1, buf_a1, buf_b1)
        return carry

    lax.fori_loop(0, NG // 2, body, 0)

    wait(NG, idx1_v, a_hbm, buf_a0, sem_a0)
    wait(NG, idx0_v, b_hbm, buf_b0, sem_b0)

    pltpu.sync_copy(out_v, out_hbm.at[wid])


@functools.partial(
    pl.kernel,
    out_type=jax.ShapeDtypeStruct((NW, NPW * O), jnp.float32),
    mesh=plsc.VectorSubcoreMesh(core_axis_name="c", subcore_axis_name="s"),
    scratch_types=[
        pltpu.VMEM((NG + 1, R), jnp.int32),
        pltpu.VMEM((NG + 1, R), jnp.int32),
        pltpu.VMEM((O,), jnp.float32),
        pltpu.VMEM((NPW * O,), jnp.float32),
        pltpu.VMEM((R, O), jnp.float32),
        pltpu.VMEM((R, O), jnp.float32),
        pltpu.VMEM((R, O), jnp.float32),
        pltpu.VMEM((R, O), jnp.float32),
        pltpu.SemaphoreType.DMA,
        pltpu.SemaphoreType.DMA,
        pltpu.SemaphoreType.DMA,
        pltpu.SemaphoreType.DMA,
    ],
)
def _sc_gather_max(a_hbm, b_hbm, idx1_hbm, idx0_hbm, bias_hbm, out_hbm,
                   *rest):
    _sc_body(a_hbm, b_hbm, idx1_hbm, idx0_hbm, bias_hbm, out_hbm, *rest)


def kernel(x, edge_index, W, b):
    xt = x[0, :, :, 0].T
    w1 = W[:, :C]
    w2 = W[:, C:]
    wa = (w1 - w2).T
    wb = w2.T

    a_tab, b_tab = _node_tables(xt, wa, wb)

    ei = edge_index.astype(jnp.int32).reshape(2, N * K)
    pad = NPAD * K - N * K
    idx1 = jnp.pad(ei[1], (0, pad)).reshape(NW, NG, R)
    idx0 = jnp.pad(ei[0], (0, pad)).reshape(NW, NG, R)
    zrow = jnp.zeros((NW, 1, R), jnp.int32)
    idx1 = jnp.concatenate([idx1, zrow], axis=1)
    idx0 = jnp.concatenate([idx0, zrow], axis=1)

    out = _sc_gather_max(a_tab, b_tab, idx1, idx0, b)
    out = out.reshape(NPAD, O)[:N].T
    return out[None]
